# baseline (speedup 1.0000x reference)
import jax
import jax.numpy as jnp
from jax.experimental import pallas as pl

N = 10000
E = 320000
G = 64


def _bias_add_kernel(x_ref, b_ref, o_ref):
    o_ref[...] = x_ref[...] + b_ref[...]


def kernel(x, edge_index, batch, edge_weight, Wr1, br1, Ws1, Wr2, br2, Ws2, Wr3, br3, Ws3, Wr4, br4, Ws4, Wr5, br5, Ws5, Wr6, br6, Ws6, Wr7, br7, Ws7, Wl, bl):
    src = edge_index[0]
    dst = edge_index[1]
    deg = jax.ops.segment_sum(jnp.ones((E,), dtype=x.dtype), dst, num_segments=N)
    has_nbr = (deg > 0)[:, None]

    def conv(h, Wr, br, Ws):
        msg = h[src] * edge_weight[:, None]
        agg = jax.ops.segment_max(msg, dst, num_segments=N)
        agg = jnp.where(has_nbr, agg, 0.0)
        return agg @ Wr + br + h @ Ws

    h = jax.nn.relu(conv(x, Wr1, br1, Ws1))
    h = jax.nn.relu(conv(h, Wr2, br2, Ws2))
    h = jax.nn.relu(conv(h, Wr3, br3, Ws3))
    h = jax.nn.relu(conv(h, Wr4, br4, Ws4))
    h = jax.nn.relu(conv(h, Wr5, br5, Ws5))
    h = jax.nn.relu(conv(h, Wr6, br6, Ws6))
    h = conv(h, Wr7, br7, Ws7)
    pooled = jax.ops.segment_sum(h, batch, num_segments=G)
    out = pooled @ Wl
    return pl.pallas_call(
        _bias_add_kernel,
        out_shape=jax.ShapeDtypeStruct(out.shape, out.dtype),
    )(out, jnp.broadcast_to(bl, out.shape))


# speedup vs baseline: 1.9302x; 1.9302x over previous
import functools

import jax
import jax.numpy as jnp
from jax import lax
from jax.experimental import pallas as pl
from jax.experimental.pallas import tpu as pltpu
from jax.experimental.pallas import tpu_sc as plsc

N = 10000
E = 320000
D = 128
G = 64
C = 10

NW = 32
NPT = 320
NPAD = NW * NPT
FL = 2048
CAP = 321536
BLK = 2000
NEG_INF = float("-inf")

_mesh = plsc.VectorSubcoreMesh(core_axis_name="c", subcore_axis_name="s")


def _wid():
    return lax.axis_index("c") * 16 + lax.axis_index("s")


def _al8(x):
    return pl.multiple_of(x, 8)


def _popcount(m):
    mi = jnp.where(m, 1, 0)
    c = mi[0]
    for q in range(1, 16):
        c = c + mi[q]
    return c


@functools.partial(
    pl.kernel,
    mesh=_mesh,
    out_type=[
        jax.ShapeDtypeStruct((NW * CAP,), jnp.int32),
        jax.ShapeDtypeStruct((NW * CAP,), jnp.float32),
        jax.ShapeDtypeStruct((NW * CAP,), jnp.int32),
        jax.ShapeDtypeStruct((NW * 8,), jnp.int32),
    ],
    scratch_types=[
        pltpu.VMEM((BLK,), jnp.int32),
        pltpu.VMEM((BLK,), jnp.int32),
        pltpu.VMEM((BLK,), jnp.float32),
        pltpu.VMEM((FL + 32,), jnp.int32),
        pltpu.VMEM((FL + 32,), jnp.float32),
        pltpu.VMEM((FL + 32,), jnp.int32),
        pltpu.VMEM((16,), jnp.int32),
    ],
)
def _partition_edges(dst_hbm, src_hbm, ew_hbm,
                     srcc_hbm, ewc_hbm, dlc_hbm, cnt_hbm,
                     dbuf, sbuf, wbuf, sb, wb, db, cvm):
    wid = _wid()
    lo = wid * NPT
    hi = lo + NPT

    def flush(co):
        cnt, off = co
        pltpu.sync_copy(sb.at[pl.ds(0, FL)], srcc_hbm.at[pl.ds(_al8(wid * CAP + off), FL)])
        pltpu.sync_copy(wb.at[pl.ds(0, FL)], ewc_hbm.at[pl.ds(_al8(wid * CAP + off), FL)])
        pltpu.sync_copy(db.at[pl.ds(0, FL)], dlc_hbm.at[pl.ds(_al8(wid * CAP + off), FL)])
        sb[pl.ds(0, 16)] = sb[pl.ds(FL, 16)]
        wb[pl.ds(0, 16)] = wb[pl.ds(FL, 16)]
        db[pl.ds(0, 16)] = db[pl.ds(FL, 16)]
        return cnt - FL, off + FL

    def block_body(b, co):
        base = _al8(b * BLK)
        pltpu.sync_copy(dst_hbm.at[pl.ds(base, BLK)], dbuf)
        pltpu.sync_copy(src_hbm.at[pl.ds(base, BLK)], sbuf)
        pltpu.sync_copy(ew_hbm.at[pl.ds(base, BLK)], wbuf)

        def vreg_body(i, co):
            d = dbuf[pl.ds(i * 16, 16)]
            m = (d >= lo) & (d < hi)

            cnt, off = co
            mi = jnp.where(m, 1, 0)
            c = mi[0]
            for q in range(1, 16):
                c = c + mi[q]

            def slow(cnt):
                s = sbuf[pl.ds(i * 16, 16)]
                w = wbuf[pl.ds(i * 16, 16)]
                dl = d - lo
                for lane in range(16):
                    def app(cnt, lane=lane):
                        sb[pl.ds(cnt, 16)] = jnp.broadcast_to(s[lane], (16,))
                        wb[pl.ds(cnt, 16)] = jnp.broadcast_to(w[lane], (16,))
                        db[pl.ds(cnt, 16)] = jnp.broadcast_to(dl[lane], (16,))
                        return cnt + 1

                    cnt = lax.cond(mi[lane] > 0, app, lambda x: x, cnt)
                return cnt

            cnt = lax.cond(c > 0, slow, lambda x: x, cnt)
            return lax.cond(cnt >= FL, flush, lambda x: x, (cnt, off))

        return lax.fori_loop(0, BLK // 16, vreg_body, co)

    cnt, off = lax.fori_loop(0, E // BLK, block_body, (0, 0))
    pltpu.sync_copy(sb.at[pl.ds(0, FL)], srcc_hbm.at[pl.ds(_al8(wid * CAP + off), FL)])
    pltpu.sync_copy(wb.at[pl.ds(0, FL)], ewc_hbm.at[pl.ds(_al8(wid * CAP + off), FL)])
    pltpu.sync_copy(db.at[pl.ds(0, FL)], dlc_hbm.at[pl.ds(_al8(wid * CAP + off), FL)])
    cvm[pl.ds(0, 16)] = jnp.broadcast_to(off + cnt, (16,))
    pltpu.sync_copy(cvm.at[pl.ds(0, 8)], cnt_hbm.at[pl.ds(_al8(wid * 8), 8)])


@functools.partial(
    pl.kernel,
    mesh=_mesh,
    out_type=jax.ShapeDtypeStruct((NPAD, D), jnp.float32),
    scratch_types=[
        pltpu.VMEM((NPT, D), jnp.float32),
        pltpu.VMEM((128,), jnp.int32),
        pltpu.VMEM((128,), jnp.float32),
        pltpu.VMEM((128,), jnp.int32),
        pltpu.VMEM((128, D), jnp.float32),
        pltpu.VMEM((16,), jnp.int32),
        pltpu.SemaphoreType.DMA,
    ],
)
def _segment_max(h_hbm, srcc_hbm, ewc_hbm, dlc_hbm, cnt_hbm, agg_hbm,
                 agg, idxb, ewb, dlb, rows, cvm, gsem):
    wid = _wid()
    pltpu.sync_copy(cnt_hbm.at[pl.ds(_al8(wid * 8), 8)], cvm.at[pl.ds(0, 8)])
    n_e = cvm[pl.ds(0, 16)][0]

    def init_row(r, u):
        for f in range(8):
            agg[r, pl.ds(f * 16, 16)] = jnp.full((16,), NEG_INF, jnp.float32)
        return u

    lax.fori_loop(0, NPT, init_row, 0)

    def chunk_body(k, u):
        base = _al8(k * 128)
        pltpu.sync_copy(srcc_hbm.at[pl.ds(_al8(wid * CAP + base), 128)], idxb)
        pltpu.sync_copy(ewc_hbm.at[pl.ds(_al8(wid * CAP + base), 128)], ewb)
        pltpu.sync_copy(dlc_hbm.at[pl.ds(_al8(wid * CAP + base), 128)], dlb)
        for v in range(8):
            iv = idxb[pl.ds(v * 16, 16)]
            idxb[pl.ds(v * 16, 16)] = jnp.clip(iv, 0, N - 1)
        pltpu.async_copy(h_hbm.at[idxb], rows, gsem).wait()
        nv = jnp.minimum(n_e - base, 128)

        def group_body(g, uu):
            gb = g * 16
            dv = dlb[pl.ds(gb, 16)]
            wv = ewb[pl.ds(gb, 16)]
            for lane in range(16):
                j = gb + lane

                @pl.when(j < nv)
                def _():
                    dloc = dv[lane]
                    w = wv[lane]
                    for f in range(8):
                        sl = pl.ds(f * 16, 16)
                        msg = rows[j, sl] * w
                        agg[dloc, sl] = jnp.maximum(agg[dloc, sl], msg)
            return uu

        lax.fori_loop(0, (nv + 15) // 16, group_body, 0)
        return u

    lax.fori_loop(0, (n_e + 127) // 128, chunk_body, 0)

    def out_row(r, u):
        for f in range(8):
            sl = pl.ds(f * 16, 16)
            v = agg[r, sl]
            agg[r, sl] = jnp.where(v == NEG_INF, 0.0, v)
        return u

    lax.fori_loop(0, NPT, out_row, 0)
    pltpu.sync_copy(agg, agg_hbm.at[pl.ds(_al8(wid * NPT), NPT)])


_RB = 1000


def _update_body(relu, agg_ref, h_ref, wr_ref, ws_ref, br_ref, o_ref):
    acc = jnp.dot(agg_ref[...], wr_ref[...], preferred_element_type=jnp.float32)
    acc = acc + jnp.dot(h_ref[...], ws_ref[...], preferred_element_type=jnp.float32)
    acc = acc + br_ref[...]
    o_ref[...] = jnp.maximum(acc, 0.0) if relu else acc


def _layer_update(agg_pad, h, wr, ws, br, relu):
    grid = N // _RB
    return pl.pallas_call(
        functools.partial(_update_body, relu),
        grid=(grid,),
        in_specs=[
            pl.BlockSpec((_RB, D), lambda k: (k, 0)),
            pl.BlockSpec((_RB, D), lambda k: (k, 0)),
            pl.BlockSpec((D, D), lambda k: (0, 0)),
            pl.BlockSpec((D, D), lambda k: (0, 0)),
            pl.BlockSpec((1, D), lambda k: (0, 0)),
        ],
        out_specs=pl.BlockSpec((_RB, D), lambda k: (k, 0)),
        out_shape=jax.ShapeDtypeStruct((N, D), jnp.float32),
    )(agg_pad, h, wr, ws, br.reshape(1, D))


def _pool_body(batch_ref, h_ref, o_ref):
    k = pl.program_id(0)
    g = lax.broadcasted_iota(jnp.int32, (G, _RB), 0)
    onehot = jnp.where(batch_ref[...].reshape(1, _RB) == g, 1.0, 0.0)
    part = jnp.dot(onehot, h_ref[...], preferred_element_type=jnp.float32)

    @pl.when(k == 0)
    def _():
        o_ref[...] = jnp.zeros_like(o_ref)

    o_ref[...] += part


def _pool(batch3d, h):
    grid = N // _RB
    return pl.pallas_call(
        _pool_body,
        grid=(grid,),
        in_specs=[
            pl.BlockSpec((1, 1, _RB), lambda k: (k, 0, 0)),
            pl.BlockSpec((_RB, D), lambda k: (k, 0)),
        ],
        out_specs=pl.BlockSpec((G, D), lambda k: (0, 0)),
        out_shape=jax.ShapeDtypeStruct((G, D), jnp.float32),
    )(batch3d, h)


def _final_body(p_ref, wl_ref, bl_ref, o_ref):
    o_ref[...] = (
        jnp.dot(p_ref[...], wl_ref[...], preferred_element_type=jnp.float32)
        + bl_ref[...]
    )


def _final(pooled, wl, bl):
    return pl.pallas_call(
        _final_body,
        out_shape=jax.ShapeDtypeStruct((G, C), jnp.float32),
    )(pooled, wl, bl.reshape(1, C))


def kernel(x, edge_index, batch, edge_weight,
           Wr1, br1, Ws1, Wr2, br2, Ws2, Wr3, br3, Ws3,
           Wr4, br4, Ws4, Wr5, br5, Ws5, Wr6, br6, Ws6,
           Wr7, br7, Ws7, Wl, bl):
    src = edge_index[0]
    dst = edge_index[1]
    srcc, ewc, dlc, cnt = _partition_edges(dst, src, edge_weight)

    layers = [
        (Wr1, br1, Ws1, True), (Wr2, br2, Ws2, True), (Wr3, br3, Ws3, True),
        (Wr4, br4, Ws4, True), (Wr5, br5, Ws5, True), (Wr6, br6, Ws6, True),
        (Wr7, br7, Ws7, False),
    ]
    h = x
    for wr, br, ws, relu in layers:
        agg_pad = _segment_max(h, srcc, ewc, dlc, cnt)
        h = _layer_update(agg_pad, h, wr, ws, br, relu)

    pooled = _pool(batch.reshape(N // _RB, 1, _RB), h)
    return _final(pooled, Wl, bl)
